# baseline (speedup 1.0000x reference)
import functools

import jax
import jax.numpy as jnp
from jax import lax
from jax.experimental import pallas as pl
from jax.experimental.pallas import tpu as pltpu
from jax.experimental.pallas import tpu_sc as plsc

NN = 50000
EE = 800000
BB = 2
NP = 51200
EP = 802816
NS = NP // 16
EPT = EP // 16
STE = 256
NST = EPT // STE
PCH = 64
NCH = NS // PCH
BLK = 1024

_mesh = plsc.VectorSubcoreMesh(core_axis_name="c", subcore_axis_name="s")
_params = pltpu.CompilerParams(use_tc_tiling_on_sc=False)


def _fill(ref, rows, width, val):
    v = jnp.full((16,), val, jnp.float32)

    def body(r, _):
        for h in range(width // 16):
            ref[r, pl.ds(h * 16, 16)] = v
        return 0

    lax.fori_loop(0, rows, body, 0)


def _wt(src, dst, sem):
    pltpu.make_async_copy(src, dst, sem).wait()


@functools.partial(
    pl.kernel,
    out_type=jax.ShapeDtypeStruct((2 * NP, 16), jnp.float32),
    mesh=_mesh,
    scratch_types=[
        pltpu.VMEM_SHARED((NP, 16), jnp.float32),
        pltpu.VMEM((512,), jnp.int32),
        pltpu.VMEM((512, 16), jnp.float32),
        pltpu.VMEM((PCH, 16), jnp.float32),
        pltpu.SemaphoreType.DMA,
    ],
    compiler_params=_params,
)
def _deg_kernel(col1d, degp, dacc, cbuf, ones, zb, sem):
    c = lax.axis_index("c")
    s = lax.axis_index("s")
    _fill(ones, 512, 16, 1.0)
    _fill(zb, PCH, 16, 0.0)

    def zero(i, _):
        pltpu.sync_copy(zb, dacc.at[pl.ds(s * NS + i * PCH, PCH)])
        return 0

    lax.fori_loop(0, NS // PCH, zero, 0)
    plsc.subcore_barrier()

    base = c * (EP // 2) + s * (EP // 32)

    def stage(t, _):
        pltpu.sync_copy(col1d.at[pl.ds(base + t * 512, 512)], cbuf)
        pltpu.async_copy(ones, dacc.at[cbuf], sem, add=True).wait()
        return 0

    lax.fori_loop(0, 49, stage, 0)
    plsc.subcore_barrier()

    def out(i, _):
        g = s * NS + i * PCH
        pltpu.sync_copy(dacc.at[pl.ds(g, PCH)], zb)
        pltpu.sync_copy(zb, degp.at[pl.ds(c * NP + g, PCH)])
        return 0

    lax.fori_loop(0, NS // PCH, out, 0)


def _make_hops(width):
    halves = width // 16

    @functools.partial(
        pl.kernel,
        out_type=(
            jax.ShapeDtypeStruct((BB * NP, width), jnp.float32),
            jax.ShapeDtypeStruct((BB * NP, width), jnp.float32),
        ),
        mesh=_mesh,
        scratch_types=[
            pltpu.VMEM_SHARED((NP, width), jnp.float32),
            pltpu.VMEM((STE,), jnp.int32),
            pltpu.VMEM((STE,), jnp.int32),
            pltpu.VMEM((STE,), jnp.int32),
            pltpu.VMEM((STE,), jnp.int32),
            pltpu.VMEM((STE, width), jnp.float32),
            pltpu.VMEM((STE, width), jnp.float32),
            pltpu.VMEM((PCH, width), jnp.float32),
            pltpu.VMEM((PCH, width), jnp.float32),
            pltpu.VMEM((PCH, width), jnp.float32),
            pltpu.VMEM((PCH, width), jnp.float32),
            pltpu.VMEM((PCH, width), jnp.float32),
            pltpu.SemaphoreType.DMA,
            pltpu.SemaphoreType.DMA,
            pltpu.SemaphoreType.DMA,
            pltpu.SemaphoreType.DMA,
            pltpu.SemaphoreType.DMA,
            pltpu.SemaphoreType.DMA,
            pltpu.SemaphoreType.DMA,
            pltpu.SemaphoreType.DMA,
            pltpu.SemaphoreType.DMA,
        ],
        compiler_params=_params,
    )
    def hops(p_init, qd2, qd1, zz, d2x, d1x, rowB, col1d, z, ptmp,
             acc, rA, cA, rB, cB, gA, gB, aA, dA, aB, dB, ibuf, isem,
             gsA, gsB, ssA, ssB, rsA, rsB, wsA, wsB):
        c = lax.axis_index("c")
        s = lax.axis_index("s")
        nbase = s * NS
        ebase = c * EP + s * EPT
        cbase = s * EPT

        def idx_load(rb, cb, t):
            pltpu.sync_copy(rowB.at[pl.ds(ebase + t * STE, STE)], rb)
            pltpu.sync_copy(col1d.at[pl.ds(cbase + t * STE, STE)], cb)

        def edge_phase(tab):
            def pair(i, _):
                t0 = 2 * i
                idx_load(rA, cA, t0)
                pltpu.async_copy(tab.at[rA], gA, gsA).wait()
                pltpu.async_copy(gA, acc.at[cA], ssA, add=True).wait()
                idx_load(rB, cB, t0 + 1)
                pltpu.async_copy(tab.at[rB], gB, gsB).wait()
                pltpu.async_copy(gB, acc.at[cB], ssB, add=True).wait()
                return 0

            lax.fori_loop(0, NST // 2, pair, 0)

        def comp(ab, db):
            def rowfn(r, _):
                for h in range(halves):
                    sl = pl.ds(h * 16, 16)
                    ab[r, sl] = ab[r, sl] * db[r, sl]
                return 0

            lax.fori_loop(0, PCH, rowfn, 0)

        def post(dref, dst, initsrc, init_off):
            ioff = (c * NP + nbase) if init_off else nbase

            def pair(j, _):
                i0 = 2 * j
                g0 = nbase + i0 * PCH
                g1 = nbase + (i0 + 1) * PCH
                pltpu.async_copy(acc.at[pl.ds(g0, PCH)], aA, rsA).wait()
                pltpu.async_copy(dref.at[pl.ds(g0, PCH)], dA, rsA).wait()
                comp(aA, dA)
                pltpu.async_copy(aA, dst.at[pl.ds(c * NP + g0, PCH)], wsA).wait()
                if initsrc is not None:
                    pltpu.async_copy(
                        initsrc.at[pl.ds(ioff + i0 * PCH, PCH)], ibuf,
                        isem).wait()
                    pltpu.sync_copy(ibuf, acc.at[pl.ds(g0, PCH)])
                pltpu.async_copy(acc.at[pl.ds(g1, PCH)], aB, rsB).wait()
                pltpu.async_copy(dref.at[pl.ds(g1, PCH)], dB, rsB).wait()
                comp(aB, dB)
                pltpu.async_copy(aB, dst.at[pl.ds(c * NP + g1, PCH)], wsB).wait()
                if initsrc is not None:
                    pltpu.async_copy(
                        initsrc.at[pl.ds(ioff + (i0 + 1) * PCH, PCH)], ibuf,
                        isem).wait()
                    pltpu.sync_copy(ibuf, acc.at[pl.ds(g1, PCH)])
                return 0

            lax.fori_loop(0, NCH // 2, pair, 0)

        def init0(i, _):
            g = nbase + i * PCH
            pltpu.async_copy(
                qd2.at[pl.ds(c * NP + g, PCH)], ibuf, isem).wait()
            pltpu.sync_copy(ibuf, acc.at[pl.ds(g, PCH)])
            return 0

        lax.fori_loop(0, NCH, init0, 0)
        plsc.subcore_barrier()
        edge_phase(p_init)
        plsc.subcore_barrier()
        post(d2x, ptmp, qd1, True)
        plsc.subcore_barrier()
        edge_phase(ptmp)
        plsc.subcore_barrier()
        post(d2x, ptmp, zz, False)
        plsc.subcore_barrier()
        edge_phase(ptmp)
        plsc.subcore_barrier()
        post(d1x, z, None, False)

    return hops


_hops32 = _make_hops(32)
_hops16 = _make_hops(16)


def _prep1_body(x_ref, degp_ref, wc_ref,
                y0, qd1, qd2, p3, d2x, d1x, d2x16, d1x16, dis_o, sdeg_o):
    deg = degp_ref[0, :, 0] + degp_ref[1, :, 0]
    dis = jnp.where(deg > 0, lax.rsqrt(jnp.maximum(deg, 1e-12)), 0.0)
    dis = dis.reshape(BLK, 1)
    sdeg = jnp.sqrt(deg).reshape(BLK, 1)
    xw = jnp.dot(x_ref[...], wc_ref[...], preferred_element_type=jnp.float32)
    y0[...] = xw[:, 0:32]
    qd1[...] = sdeg * xw[:, 32:64]
    qd2[...] = sdeg * xw[:, 64:96]
    p3[...] = dis * xw[:, 96:128]
    d2 = dis * dis
    d2x[...] = jnp.broadcast_to(d2, (BLK, 32))
    d1x[...] = jnp.broadcast_to(dis, (BLK, 32))
    d2x16[...] = jnp.broadcast_to(d2, (BLK, 16))
    d1x16[...] = jnp.broadcast_to(dis, (BLK, 16))
    dis_o[...] = dis
    sdeg_o[...] = sdeg


def _prep1(xf, degp2, wc1):
    nb = NP // BLK
    fo32 = jax.ShapeDtypeStruct((BB * NP, 32), jnp.float32)
    n32 = jax.ShapeDtypeStruct((NP, 32), jnp.float32)
    n16 = jax.ShapeDtypeStruct((NP, 16), jnp.float32)
    n1 = jax.ShapeDtypeStruct((NP, 1), jnp.float32)
    bs_bn = pl.BlockSpec((BLK, 32), lambda i: (i, 0))
    bs_n32 = pl.BlockSpec((BLK, 32), lambda i: (i % nb, 0))
    bs_n16 = pl.BlockSpec((BLK, 16), lambda i: (i % nb, 0))
    bs_n1 = pl.BlockSpec((BLK, 1), lambda i: (i % nb, 0))
    return pl.pallas_call(
        _prep1_body,
        grid=(BB * NP // BLK,),
        in_specs=[
            bs_bn,
            pl.BlockSpec((2, BLK, 16), lambda i: (0, i % nb, 0)),
            pl.BlockSpec((32, 128), lambda i: (0, 0)),
        ],
        out_specs=[bs_bn, bs_bn, bs_bn, bs_bn,
                   bs_n32, bs_n32, bs_n16, bs_n16, bs_n1, bs_n1],
        out_shape=[fo32, fo32, fo32, fo32, n32, n32, n16, n16, n1, n1],
    )(xf, degp2, wc1)


def _mid_body(wo, y0_ref, z_ref, g_ref, be_ref, dis_ref, sdeg_ref, b_ref,
              wc_ref, y0n, qd1n, qd2n, p3n):
    fo = wo // 4
    t = y0_ref[...] + z_ref[...] + b_ref[...]
    m = jnp.mean(t, axis=0, keepdims=True)
    v = jnp.mean((t - m) ** 2, axis=0, keepdims=True)
    h = g_ref[...] * (t - m) / jnp.sqrt(v + 1e-5) + be_ref[...]
    h = jnp.where(h >= 0, h, 0.01 * h)
    hw = jnp.dot(h.reshape(BB * BLK, 32), wc_ref[...],
                 preferred_element_type=jnp.float32).reshape(BB, BLK, wo)
    dis = dis_ref[...].reshape(1, BLK, 1)
    sdeg = sdeg_ref[...].reshape(1, BLK, 1)
    y0n[...] = hw[..., 0:fo]
    qd1n[...] = sdeg * hw[..., fo:2 * fo]
    qd2n[...] = sdeg * hw[..., 2 * fo:3 * fo]
    p3n[...] = dis * hw[..., 3 * fo:4 * fo]


def _mid(y0, zz, gp, bep, dis, sdeg, br, wc):
    wo = wc.shape[1]
    fo = wo // 4
    out = jax.ShapeDtypeStruct((BB, NP, fo), jnp.float32)
    bs_o = pl.BlockSpec((BB, BLK, fo), lambda i: (0, i, 0))
    return pl.pallas_call(
        functools.partial(_mid_body, wo),
        grid=(NP // BLK,),
        in_specs=[
            pl.BlockSpec((BB, BLK, 32), lambda i: (0, i, 0)),
            pl.BlockSpec((BB, BLK, 32), lambda i: (0, i, 0)),
            pl.BlockSpec((BLK, 32), lambda i: (i, 0)),
            pl.BlockSpec((BLK, 32), lambda i: (i, 0)),
            pl.BlockSpec((BLK, 1), lambda i: (i, 0)),
            pl.BlockSpec((BLK, 1), lambda i: (i, 0)),
            pl.BlockSpec((1, 32), lambda i: (0, 0)),
            pl.BlockSpec((32, wo), lambda i: (0, 0)),
        ],
        out_specs=[bs_o, bs_o, bs_o, bs_o],
        out_shape=[out, out, out, out],
    )(y0, zz, gp, bep, dis, sdeg, br, wc)


def _final_body(y0_ref, z_ref, b_ref, vmin_ref, vmax_ref, o_ref):
    t = y0_ref[...] + z_ref[...] + b_ref[...]
    sg = jax.nn.sigmoid(t * 0.1)
    a = vmin_ref[...][None]
    b = vmax_ref[...][None]
    o_ref[...] = a + (b - a) * sg


def _final(y0, zz, b3r, vminp, vmaxp):
    return pl.pallas_call(
        _final_body,
        grid=(NP // BLK,),
        in_specs=[
            pl.BlockSpec((BB, BLK, 16), lambda i: (0, i, 0)),
            pl.BlockSpec((BB, BLK, 16), lambda i: (0, i, 0)),
            pl.BlockSpec((1, 16), lambda i: (0, 0)),
            pl.BlockSpec((BLK, 16), lambda i: (i, 0)),
            pl.BlockSpec((BLK, 16), lambda i: (i, 0)),
        ],
        out_specs=pl.BlockSpec((BB, BLK, 16), lambda i: (0, i, 0)),
        out_shape=jax.ShapeDtypeStruct((BB, NP, 16), jnp.float32),
    )(y0, zz, b3r, vminp, vmaxp)


@jax.jit
def kernel(x, edge_index, W1, b1, W2, b2, W3, b3, g1, be1, g2, be2,
           val_min, val_max):
    row = edge_index[0]
    col = edge_index[1]
    rowp = jnp.concatenate([row, jnp.full((EP - EE,), NN, jnp.int32)])
    colp = jnp.concatenate([col, jnp.full((EP - EE,), NN + 1, jnp.int32)])
    rowB = jnp.concatenate([rowp, rowp + NP])
    col1d = colp

    xf = jnp.pad(x, ((0, 0), (0, NP - NN), (0, 0))).reshape(BB * NP, 32)
    wc1 = jnp.concatenate([W1[0], W1[1], W1[2], W1[3]], axis=1)
    wc2 = jnp.concatenate([W2[0], W2[1], W2[2], W2[3]], axis=1)
    w3p = jnp.pad(W3, ((0, 0), (0, 0), (0, 13)))
    wc3 = jnp.concatenate([w3p[0], w3p[1], w3p[2], w3p[3]], axis=1)
    g1p = jnp.pad(g1.reshape(NN, 32), ((0, NP - NN), (0, 0)))
    be1p = jnp.pad(be1.reshape(NN, 32), ((0, NP - NN), (0, 0)))
    g2p = jnp.pad(g2.reshape(NN, 32), ((0, NP - NN), (0, 0)))
    be2p = jnp.pad(be2.reshape(NN, 32), ((0, NP - NN), (0, 0)))
    vminp = jnp.pad(val_min, ((0, NP - NN), (0, 13)))
    vmaxp = jnp.pad(val_max, ((0, NP - NN), (0, 13)))
    b1r = b1.reshape(1, 32)
    b2r = b2.reshape(1, 32)
    b3r = jnp.pad(b3, (0, 13)).reshape(1, 16)
    z32 = jnp.zeros((NP, 32), jnp.float32)
    z16 = jnp.zeros((NP, 16), jnp.float32)

    degp = _deg_kernel(col1d)
    y0a, qd1a, qd2a, p3a, d2x, d1x, d2x16, d1x16, dis, sdeg = _prep1(
        xf, degp.reshape(2, NP, 16), wc1)

    z1, _ = _hops32(p3a, qd2a, qd1a, z32, d2x, d1x, rowB, col1d)

    y0b, qd1b, qd2b, p3b = _mid(y0a.reshape(BB, NP, 32),
                                z1.reshape(BB, NP, 32),
                                g1p, be1p, dis, sdeg, b1r, wc2)
    z2, _ = _hops32(p3b.reshape(BB * NP, 32), qd2b.reshape(BB * NP, 32),
                    qd1b.reshape(BB * NP, 32), z32, d2x, d1x, rowB, col1d)

    y0c, qd1c, qd2c, p3c = _mid(y0b, z2.reshape(BB, NP, 32),
                                g2p, be2p, dis, sdeg, b2r, wc3)
    z3, _ = _hops16(p3c.reshape(BB * NP, 16), qd2c.reshape(BB * NP, 16),
                    qd1c.reshape(BB * NP, 16), z16, d2x16, d1x16, rowB, col1d)

    res = _final(y0c, z3.reshape(BB, NP, 16), b3r, vminp, vmaxp)
    return res[:, :NN, :3]


# speedup vs baseline: 1.2006x; 1.2006x over previous
import functools

import jax
import jax.numpy as jnp
from jax import lax
from jax.experimental import pallas as pl
from jax.experimental.pallas import tpu as pltpu
from jax.experimental.pallas import tpu_sc as plsc

NN = 50000
EE = 800000
BB = 2
NP = 51200
EP = 802816
NS = NP // 16
EPT = EP // 16
STE = 256
NST = EPT // STE
PCH = 64
NCH = NS // PCH
BLK = 1024

_mesh = plsc.VectorSubcoreMesh(core_axis_name="c", subcore_axis_name="s")
_params = pltpu.CompilerParams(use_tc_tiling_on_sc=False)


def _fill(ref, rows, width, val):
    v = jnp.full((16,), val, jnp.float32)

    def body(r, _):
        for h in range(width // 16):
            ref[r, pl.ds(h * 16, 16)] = v
        return 0

    lax.fori_loop(0, rows, body, 0)


def _wt(src, dst, sem):
    pltpu.make_async_copy(src, dst, sem).wait()


@functools.partial(
    pl.kernel,
    out_type=jax.ShapeDtypeStruct((2 * NP, 16), jnp.float32),
    mesh=_mesh,
    scratch_types=[
        pltpu.VMEM_SHARED((NP, 16), jnp.float32),
        pltpu.VMEM((512,), jnp.int32),
        pltpu.VMEM((512, 16), jnp.float32),
        pltpu.VMEM((PCH, 16), jnp.float32),
        pltpu.SemaphoreType.DMA,
    ],
    compiler_params=_params,
)
def _deg_kernel(col1d, degp, dacc, cbuf, ones, zb, sem):
    c = lax.axis_index("c")
    s = lax.axis_index("s")
    _fill(ones, 512, 16, 1.0)
    _fill(zb, PCH, 16, 0.0)

    def zero(i, _):
        pltpu.sync_copy(zb, dacc.at[pl.ds(s * NS + i * PCH, PCH)])
        return 0

    lax.fori_loop(0, NS // PCH, zero, 0)
    plsc.subcore_barrier()

    base = c * (EP // 2) + s * (EP // 32)

    def stage(t, _):
        pltpu.sync_copy(col1d.at[pl.ds(base + t * 512, 512)], cbuf)
        pltpu.async_copy(ones, dacc.at[cbuf], sem, add=True).wait()
        return 0

    lax.fori_loop(0, 49, stage, 0)
    plsc.subcore_barrier()

    def out(i, _):
        g = s * NS + i * PCH
        pltpu.sync_copy(dacc.at[pl.ds(g, PCH)], zb)
        pltpu.sync_copy(zb, degp.at[pl.ds(c * NP + g, PCH)])
        return 0

    lax.fori_loop(0, NS // PCH, out, 0)


def _make_hops(width):
    halves = width // 16

    @functools.partial(
        pl.kernel,
        out_type=(
            jax.ShapeDtypeStruct((BB * NP, width), jnp.float32),
            jax.ShapeDtypeStruct((BB * NP, width), jnp.float32),
        ),
        mesh=_mesh,
        scratch_types=[
            pltpu.VMEM_SHARED((NP, width), jnp.float32),
            pltpu.VMEM((STE,), jnp.int32),
            pltpu.VMEM((STE,), jnp.int32),
            pltpu.VMEM((STE,), jnp.int32),
            pltpu.VMEM((STE,), jnp.int32),
            pltpu.VMEM((STE, width), jnp.float32),
            pltpu.VMEM((STE, width), jnp.float32),
            pltpu.VMEM((PCH, width), jnp.float32),
            pltpu.VMEM((PCH, width), jnp.float32),
            pltpu.VMEM((PCH, width), jnp.float32),
            pltpu.VMEM((PCH, width), jnp.float32),
            pltpu.VMEM((PCH, width), jnp.float32),
            pltpu.SemaphoreType.DMA,
            pltpu.SemaphoreType.DMA,
            pltpu.SemaphoreType.DMA,
            pltpu.SemaphoreType.DMA,
            pltpu.SemaphoreType.DMA,
            pltpu.SemaphoreType.DMA,
            pltpu.SemaphoreType.DMA,
            pltpu.SemaphoreType.DMA,
            pltpu.SemaphoreType.DMA,
        ],
        compiler_params=_params,
    )
    def hops(p_init, qd2, qd1, zz, d2x, d1x, rowB, col1d, z, ptmp,
             acc, rA, cA, rB, cB, gA, gB, aA, dA, aB, dB, ibuf, isem,
             gsA, gsB, ssA, ssB, rsA, rsB, wsA, wsB):
        c = lax.axis_index("c")
        s = lax.axis_index("s")
        nbase = s * NS
        ebase = c * EP + s * EPT
        cbase = s * EPT

        def idx_load(rb, cb, t):
            pltpu.sync_copy(rowB.at[pl.ds(ebase + t * STE, STE)], rb)
            pltpu.sync_copy(col1d.at[pl.ds(cbase + t * STE, STE)], cb)

        def edge_phase(tab):
            def pair(i, _):
                t0 = 2 * i
                idx_load(rA, cA, t0)
                hga = pltpu.async_copy(tab.at[rA], gA, gsA)
                idx_load(rB, cB, t0 + 1)
                hgb = pltpu.async_copy(tab.at[rB], gB, gsB)
                hga.wait()
                hsa = pltpu.async_copy(gA, acc.at[cA], ssA, add=True)
                hgb.wait()
                hsb = pltpu.async_copy(gB, acc.at[cB], ssB, add=True)
                hsa.wait()
                hsb.wait()
                return 0

            lax.fori_loop(0, NST // 2, pair, 0)

        def comp(ab, db):
            def rowfn(r, _):
                for h in range(halves):
                    sl = pl.ds(h * 16, 16)
                    ab[r, sl] = ab[r, sl] * db[r, sl]
                return 0

            lax.fori_loop(0, PCH, rowfn, 0)

        def post(dref, dst, initsrc, init_off):
            ioff = (c * NP + nbase) if init_off else nbase

            def pair(j, _):
                i0 = 2 * j
                g0 = nbase + i0 * PCH
                g1 = nbase + (i0 + 1) * PCH
                pltpu.async_copy(acc.at[pl.ds(g0, PCH)], aA, rsA).wait()
                pltpu.async_copy(dref.at[pl.ds(g0, PCH)], dA, rsA).wait()
                comp(aA, dA)
                pltpu.async_copy(aA, dst.at[pl.ds(c * NP + g0, PCH)], wsA).wait()
                if initsrc is not None:
                    pltpu.async_copy(
                        initsrc.at[pl.ds(ioff + i0 * PCH, PCH)], ibuf,
                        isem).wait()
                    pltpu.sync_copy(ibuf, acc.at[pl.ds(g0, PCH)])
                pltpu.async_copy(acc.at[pl.ds(g1, PCH)], aB, rsB).wait()
                pltpu.async_copy(dref.at[pl.ds(g1, PCH)], dB, rsB).wait()
                comp(aB, dB)
                pltpu.async_copy(aB, dst.at[pl.ds(c * NP + g1, PCH)], wsB).wait()
                if initsrc is not None:
                    pltpu.async_copy(
                        initsrc.at[pl.ds(ioff + (i0 + 1) * PCH, PCH)], ibuf,
                        isem).wait()
                    pltpu.sync_copy(ibuf, acc.at[pl.ds(g1, PCH)])
                return 0

            lax.fori_loop(0, NCH // 2, pair, 0)

        def init0(i, _):
            g = nbase + i * PCH
            pltpu.async_copy(
                qd2.at[pl.ds(c * NP + g, PCH)], ibuf, isem).wait()
            pltpu.sync_copy(ibuf, acc.at[pl.ds(g, PCH)])
            return 0

        lax.fori_loop(0, NCH, init0, 0)
        plsc.subcore_barrier()
        edge_phase(p_init)
        plsc.subcore_barrier()
        post(d2x, ptmp, qd1, True)
        plsc.subcore_barrier()
        edge_phase(ptmp)
        plsc.subcore_barrier()
        post(d2x, ptmp, zz, False)
        plsc.subcore_barrier()
        edge_phase(ptmp)
        plsc.subcore_barrier()
        post(d1x, z, None, False)

    return hops


_hops32 = _make_hops(32)
_hops16 = _make_hops(16)


def _prep1_body(x_ref, degp_ref, wc_ref,
                y0, qd1, qd2, p3, d2x, d1x, d2x16, d1x16, dis_o, sdeg_o):
    deg = degp_ref[0, :, 0] + degp_ref[1, :, 0]
    dis = jnp.where(deg > 0, lax.rsqrt(jnp.maximum(deg, 1e-12)), 0.0)
    dis = dis.reshape(BLK, 1)
    sdeg = jnp.sqrt(deg).reshape(BLK, 1)
    xw = jnp.dot(x_ref[...], wc_ref[...], preferred_element_type=jnp.float32)
    y0[...] = xw[:, 0:32]
    qd1[...] = sdeg * xw[:, 32:64]
    qd2[...] = sdeg * xw[:, 64:96]
    p3[...] = dis * xw[:, 96:128]
    d2 = dis * dis
    d2x[...] = jnp.broadcast_to(d2, (BLK, 32))
    d1x[...] = jnp.broadcast_to(dis, (BLK, 32))
    d2x16[...] = jnp.broadcast_to(d2, (BLK, 16))
    d1x16[...] = jnp.broadcast_to(dis, (BLK, 16))
    dis_o[...] = dis
    sdeg_o[...] = sdeg


def _prep1(xf, degp2, wc1):
    nb = NP // BLK
    fo32 = jax.ShapeDtypeStruct((BB * NP, 32), jnp.float32)
    n32 = jax.ShapeDtypeStruct((NP, 32), jnp.float32)
    n16 = jax.ShapeDtypeStruct((NP, 16), jnp.float32)
    n1 = jax.ShapeDtypeStruct((NP, 1), jnp.float32)
    bs_bn = pl.BlockSpec((BLK, 32), lambda i: (i, 0))
    bs_n32 = pl.BlockSpec((BLK, 32), lambda i: (i % nb, 0))
    bs_n16 = pl.BlockSpec((BLK, 16), lambda i: (i % nb, 0))
    bs_n1 = pl.BlockSpec((BLK, 1), lambda i: (i % nb, 0))
    return pl.pallas_call(
        _prep1_body,
        grid=(BB * NP // BLK,),
        in_specs=[
            bs_bn,
            pl.BlockSpec((2, BLK, 16), lambda i: (0, i % nb, 0)),
            pl.BlockSpec((32, 128), lambda i: (0, 0)),
        ],
        out_specs=[bs_bn, bs_bn, bs_bn, bs_bn,
                   bs_n32, bs_n32, bs_n16, bs_n16, bs_n1, bs_n1],
        out_shape=[fo32, fo32, fo32, fo32, n32, n32, n16, n16, n1, n1],
    )(xf, degp2, wc1)


def _mid_body(wo, y0_ref, z_ref, g_ref, be_ref, dis_ref, sdeg_ref, b_ref,
              wc_ref, y0n, qd1n, qd2n, p3n):
    fo = wo // 4
    t = y0_ref[...] + z_ref[...] + b_ref[...]
    m = jnp.mean(t, axis=0, keepdims=True)
    v = jnp.mean((t - m) ** 2, axis=0, keepdims=True)
    h = g_ref[...] * (t - m) / jnp.sqrt(v + 1e-5) + be_ref[...]
    h = jnp.where(h >= 0, h, 0.01 * h)
    hw = jnp.dot(h.reshape(BB * BLK, 32), wc_ref[...],
                 preferred_element_type=jnp.float32).reshape(BB, BLK, wo)
    dis = dis_ref[...].reshape(1, BLK, 1)
    sdeg = sdeg_ref[...].reshape(1, BLK, 1)
    y0n[...] = hw[..., 0:fo]
    qd1n[...] = sdeg * hw[..., fo:2 * fo]
    qd2n[...] = sdeg * hw[..., 2 * fo:3 * fo]
    p3n[...] = dis * hw[..., 3 * fo:4 * fo]


def _mid(y0, zz, gp, bep, dis, sdeg, br, wc):
    wo = wc.shape[1]
    fo = wo // 4
    out = jax.ShapeDtypeStruct((BB, NP, fo), jnp.float32)
    bs_o = pl.BlockSpec((BB, BLK, fo), lambda i: (0, i, 0))
    return pl.pallas_call(
        functools.partial(_mid_body, wo),
        grid=(NP // BLK,),
        in_specs=[
            pl.BlockSpec((BB, BLK, 32), lambda i: (0, i, 0)),
            pl.BlockSpec((BB, BLK, 32), lambda i: (0, i, 0)),
            pl.BlockSpec((BLK, 32), lambda i: (i, 0)),
            pl.BlockSpec((BLK, 32), lambda i: (i, 0)),
            pl.BlockSpec((BLK, 1), lambda i: (i, 0)),
            pl.BlockSpec((BLK, 1), lambda i: (i, 0)),
            pl.BlockSpec((1, 32), lambda i: (0, 0)),
            pl.BlockSpec((32, wo), lambda i: (0, 0)),
        ],
        out_specs=[bs_o, bs_o, bs_o, bs_o],
        out_shape=[out, out, out, out],
    )(y0, zz, gp, bep, dis, sdeg, br, wc)


def _final_body(y0_ref, z_ref, b_ref, vmin_ref, vmax_ref, o_ref):
    t = y0_ref[...] + z_ref[...] + b_ref[...]
    sg = jax.nn.sigmoid(t * 0.1)
    a = vmin_ref[...][None]
    b = vmax_ref[...][None]
    o_ref[...] = a + (b - a) * sg


def _final(y0, zz, b3r, vminp, vmaxp):
    return pl.pallas_call(
        _final_body,
        grid=(NP // BLK,),
        in_specs=[
            pl.BlockSpec((BB, BLK, 16), lambda i: (0, i, 0)),
            pl.BlockSpec((BB, BLK, 16), lambda i: (0, i, 0)),
            pl.BlockSpec((1, 16), lambda i: (0, 0)),
            pl.BlockSpec((BLK, 16), lambda i: (i, 0)),
            pl.BlockSpec((BLK, 16), lambda i: (i, 0)),
        ],
        out_specs=pl.BlockSpec((BB, BLK, 16), lambda i: (0, i, 0)),
        out_shape=jax.ShapeDtypeStruct((BB, NP, 16), jnp.float32),
    )(y0, zz, b3r, vminp, vmaxp)


@jax.jit
def kernel(x, edge_index, W1, b1, W2, b2, W3, b3, g1, be1, g2, be2,
           val_min, val_max):
    row = edge_index[0]
    col = edge_index[1]
    rowp = jnp.concatenate([row, jnp.full((EP - EE,), NN, jnp.int32)])
    colp = jnp.concatenate([col, jnp.full((EP - EE,), NN + 1, jnp.int32)])
    rowB = jnp.concatenate([rowp, rowp + NP])
    col1d = colp

    xf = jnp.pad(x, ((0, 0), (0, NP - NN), (0, 0))).reshape(BB * NP, 32)
    wc1 = jnp.concatenate([W1[0], W1[1], W1[2], W1[3]], axis=1)
    wc2 = jnp.concatenate([W2[0], W2[1], W2[2], W2[3]], axis=1)
    w3p = jnp.pad(W3, ((0, 0), (0, 0), (0, 13)))
    wc3 = jnp.concatenate([w3p[0], w3p[1], w3p[2], w3p[3]], axis=1)
    g1p = jnp.pad(g1.reshape(NN, 32), ((0, NP - NN), (0, 0)))
    be1p = jnp.pad(be1.reshape(NN, 32), ((0, NP - NN), (0, 0)))
    g2p = jnp.pad(g2.reshape(NN, 32), ((0, NP - NN), (0, 0)))
    be2p = jnp.pad(be2.reshape(NN, 32), ((0, NP - NN), (0, 0)))
    vminp = jnp.pad(val_min, ((0, NP - NN), (0, 13)))
    vmaxp = jnp.pad(val_max, ((0, NP - NN), (0, 13)))
    b1r = b1.reshape(1, 32)
    b2r = b2.reshape(1, 32)
    b3r = jnp.pad(b3, (0, 13)).reshape(1, 16)
    z32 = jnp.zeros((NP, 32), jnp.float32)
    z16 = jnp.zeros((NP, 16), jnp.float32)

    degp = _deg_kernel(col1d)
    y0a, qd1a, qd2a, p3a, d2x, d1x, d2x16, d1x16, dis, sdeg = _prep1(
        xf, degp.reshape(2, NP, 16), wc1)

    z1, _ = _hops32(p3a, qd2a, qd1a, z32, d2x, d1x, rowB, col1d)

    y0b, qd1b, qd2b, p3b = _mid(y0a.reshape(BB, NP, 32),
                                z1.reshape(BB, NP, 32),
                                g1p, be1p, dis, sdeg, b1r, wc2)
    z2, _ = _hops32(p3b.reshape(BB * NP, 32), qd2b.reshape(BB * NP, 32),
                    qd1b.reshape(BB * NP, 32), z32, d2x, d1x, rowB, col1d)

    y0c, qd1c, qd2c, p3c = _mid(y0b, z2.reshape(BB, NP, 32),
                                g2p, be2p, dis, sdeg, b2r, wc3)
    z3, _ = _hops16(p3c.reshape(BB * NP, 16), qd2c.reshape(BB * NP, 16),
                    qd1c.reshape(BB * NP, 16), z16, d2x16, d1x16, rowB, col1d)

    res = _final(y0c, z3.reshape(BB, NP, 16), b3r, vminp, vmaxp)
    return res[:, :NN, :3]


# speedup vs baseline: 1.3900x; 1.1577x over previous
import functools

import jax
import jax.numpy as jnp
from jax import lax
from jax.experimental import pallas as pl
from jax.experimental.pallas import tpu as pltpu
from jax.experimental.pallas import tpu_sc as plsc

NN = 50000
EE = 800000
BB = 2
NP = 51200
EP = 802816
NS = NP // 16
EPT = EP // 16
STE = 256
NST = EPT // STE
PCH = 64
NCH = NS // PCH
BLK = 1024

_mesh = plsc.VectorSubcoreMesh(core_axis_name="c", subcore_axis_name="s")
_params = pltpu.CompilerParams(use_tc_tiling_on_sc=False)


def _fill(ref, rows, width, val):
    v = jnp.full((16,), val, jnp.float32)

    def body(r, _):
        for h in range(width // 16):
            ref[r, pl.ds(h * 16, 16)] = v
        return 0

    lax.fori_loop(0, rows, body, 0)


def _wt(src, dst, sem):
    pltpu.make_async_copy(src, dst, sem).wait()


@functools.partial(
    pl.kernel,
    out_type=jax.ShapeDtypeStruct((2 * NP, 16), jnp.float32),
    mesh=_mesh,
    scratch_types=[
        pltpu.VMEM_SHARED((NP, 16), jnp.float32),
        pltpu.VMEM((512,), jnp.int32),
        pltpu.VMEM((512, 16), jnp.float32),
        pltpu.VMEM((PCH, 16), jnp.float32),
        pltpu.SemaphoreType.DMA,
    ],
    compiler_params=_params,
)
def _deg_kernel(col1d, degp, dacc, cbuf, ones, zb, sem):
    c = lax.axis_index("c")
    s = lax.axis_index("s")
    _fill(ones, 512, 16, 1.0)
    _fill(zb, PCH, 16, 0.0)

    def zero(i, _):
        pltpu.sync_copy(zb, dacc.at[pl.ds(s * NS + i * PCH, PCH)])
        return 0

    lax.fori_loop(0, NS // PCH, zero, 0)
    plsc.subcore_barrier()

    base = c * (EP // 2) + s * (EP // 32)

    def stage(t, _):
        pltpu.sync_copy(col1d.at[pl.ds(base + t * 512, 512)], cbuf)
        pltpu.async_copy(ones, dacc.at[cbuf], sem, add=True).wait()
        return 0

    lax.fori_loop(0, 49, stage, 0)
    plsc.subcore_barrier()

    def out(i, _):
        g = s * NS + i * PCH
        pltpu.sync_copy(dacc.at[pl.ds(g, PCH)], zb)
        pltpu.sync_copy(zb, degp.at[pl.ds(c * NP + g, PCH)])
        return 0

    lax.fori_loop(0, NS // PCH, out, 0)


CPT = EP // 128 // 16
PPC = 128


def _make_hops(width):
    halves = width // 16

    @functools.partial(
        pl.kernel,
        out_type=(
            jax.ShapeDtypeStruct((BB * NP, width), jnp.float32),
            jax.ShapeDtypeStruct((BB * NP, width), jnp.float32),
        ),
        mesh=_mesh,
        scratch_types=[
            pltpu.VMEM_SHARED((NP, width), jnp.float32),
            pltpu.VMEM((512,), jnp.int32),
            pltpu.VMEM((4, 128), jnp.int32),
            pltpu.VMEM((128, width), jnp.float32),
            pltpu.VMEM((128, width), jnp.float32),
            pltpu.VMEM((128, width), jnp.float32),
            pltpu.VMEM((128, width), jnp.float32),
            pltpu.SemaphoreType.DMA,
            pltpu.SemaphoreType.DMA,
            pltpu.SemaphoreType.DMA,
            pltpu.SemaphoreType.DMA,
            pltpu.SemaphoreType.DMA,
            pltpu.SemaphoreType.DMA,
            pltpu.SemaphoreType.DMA,
            pltpu.SemaphoreType.DMA,
            pltpu.SemaphoreType.DMA,
        ],
        compiler_params=_params,
    )
    def hops(p_init, qd2, qd1, zz, d2x, d1x, rowB, col2d, z, ptmp,
             acc, rbig, cbig, gb0, gb1, gb2, gb3,
             gs0, gs1, gs2, gs3, ss0, ss1, ss2, ss3, isem):
        c = lax.axis_index("c")
        s = lax.axis_index("s")
        nbase = s * NS
        ebase = c * EP + s * EPT
        gbs = [gb0, gb1, gb2, gb3]
        gss = [gs0, gs1, gs2, gs3]
        sss = [ss0, ss1, ss2, ss3]

        def edge_phase(tab):
            def body(i, _):
                pltpu.sync_copy(rowB.at[pl.ds(ebase + i * 512, 512)], rbig)
                pltpu.sync_copy(col2d.at[pl.ds(s * CPT + i * 4, 4)], cbig)
                hg = [
                    pltpu.async_copy(
                        tab.at[rbig.at[pl.ds(j * 128, 128)]], gbs[j], gss[j])
                    for j in range(4)
                ]
                hs = []
                for j in range(4):
                    hg[j].wait()
                    hs.append(pltpu.async_copy(
                        gbs[j], acc.at[cbig.at[j]], sss[j], add=True))
                for h in hs:
                    h.wait()
                return 0

            lax.fori_loop(0, EPT // 512, body, 0)

        def comp(ab, db):
            def rowfn(r, _):
                for h in range(halves):
                    sl = pl.ds(h * 16, 16)
                    ab[r, sl] = ab[r, sl] * db[r, sl]
                return 0

            lax.fori_loop(0, PPC, rowfn, 0)

        def post(dref, dst, initsrc, init_off):
            ioff = (c * NP + nbase) if init_off else nbase

            def chunk(i, _):
                g = nbase + i * PPC
                pltpu.async_copy(acc.at[pl.ds(g, PPC)], gb0, gs0).wait()
                pltpu.async_copy(dref.at[pl.ds(g, PPC)], gb1, gs1).wait()
                comp(gb0, gb1)
                pltpu.async_copy(
                    gb0, dst.at[pl.ds(c * NP + g, PPC)], ss0).wait()
                if initsrc is not None:
                    pltpu.async_copy(
                        initsrc.at[pl.ds(ioff + i * PPC, PPC)], gb2,
                        isem).wait()
                    pltpu.sync_copy(gb2, acc.at[pl.ds(g, PPC)])
                return 0

            lax.fori_loop(0, NS // PPC, chunk, 0)

        def init0(i, _):
            g = nbase + i * PPC
            pltpu.async_copy(
                qd2.at[pl.ds(c * NP + g, PPC)], gb2, isem).wait()
            pltpu.sync_copy(gb2, acc.at[pl.ds(g, PPC)])
            return 0

        lax.fori_loop(0, NS // PPC, init0, 0)
        plsc.subcore_barrier()
        edge_phase(p_init)
        plsc.subcore_barrier()
        post(d2x, ptmp, qd1, True)
        plsc.subcore_barrier()
        edge_phase(ptmp)
        plsc.subcore_barrier()
        post(d2x, ptmp, zz, False)
        plsc.subcore_barrier()
        edge_phase(ptmp)
        plsc.subcore_barrier()
        post(d1x, z, None, False)

    return hops


_hops32 = _make_hops(32)
_hops16 = _make_hops(16)


def _prep1_body(x_ref, degp_ref, wc_ref,
                y0, qd1, qd2, p3, d2x, d1x, d2x16, d1x16, dis_o, sdeg_o):
    deg = degp_ref[0, :, 0] + degp_ref[1, :, 0]
    dis = jnp.where(deg > 0, lax.rsqrt(jnp.maximum(deg, 1e-12)), 0.0)
    dis = dis.reshape(BLK, 1)
    sdeg = jnp.sqrt(deg).reshape(BLK, 1)
    xw = jnp.dot(x_ref[...], wc_ref[...], preferred_element_type=jnp.float32)
    y0[...] = xw[:, 0:32]
    qd1[...] = sdeg * xw[:, 32:64]
    qd2[...] = sdeg * xw[:, 64:96]
    p3[...] = dis * xw[:, 96:128]
    d2 = dis * dis
    d2x[...] = jnp.broadcast_to(d2, (BLK, 32))
    d1x[...] = jnp.broadcast_to(dis, (BLK, 32))
    d2x16[...] = jnp.broadcast_to(d2, (BLK, 16))
    d1x16[...] = jnp.broadcast_to(dis, (BLK, 16))
    dis_o[...] = dis
    sdeg_o[...] = sdeg


def _prep1(xf, degp2, wc1):
    nb = NP // BLK
    fo32 = jax.ShapeDtypeStruct((BB * NP, 32), jnp.float32)
    n32 = jax.ShapeDtypeStruct((NP, 32), jnp.float32)
    n16 = jax.ShapeDtypeStruct((NP, 16), jnp.float32)
    n1 = jax.ShapeDtypeStruct((NP, 1), jnp.float32)
    bs_bn = pl.BlockSpec((BLK, 32), lambda i: (i, 0))
    bs_n32 = pl.BlockSpec((BLK, 32), lambda i: (i % nb, 0))
    bs_n16 = pl.BlockSpec((BLK, 16), lambda i: (i % nb, 0))
    bs_n1 = pl.BlockSpec((BLK, 1), lambda i: (i % nb, 0))
    return pl.pallas_call(
        _prep1_body,
        grid=(BB * NP // BLK,),
        in_specs=[
            bs_bn,
            pl.BlockSpec((2, BLK, 16), lambda i: (0, i % nb, 0)),
            pl.BlockSpec((32, 128), lambda i: (0, 0)),
        ],
        out_specs=[bs_bn, bs_bn, bs_bn, bs_bn,
                   bs_n32, bs_n32, bs_n16, bs_n16, bs_n1, bs_n1],
        out_shape=[fo32, fo32, fo32, fo32, n32, n32, n16, n16, n1, n1],
    )(xf, degp2, wc1)


def _mid_body(wo, y0_ref, z_ref, g_ref, be_ref, dis_ref, sdeg_ref, b_ref,
              wc_ref, y0n, qd1n, qd2n, p3n):
    fo = wo // 4
    t = y0_ref[...] + z_ref[...] + b_ref[...]
    m = jnp.mean(t, axis=0, keepdims=True)
    v = jnp.mean((t - m) ** 2, axis=0, keepdims=True)
    h = g_ref[...] * (t - m) / jnp.sqrt(v + 1e-5) + be_ref[...]
    h = jnp.where(h >= 0, h, 0.01 * h)
    hw = jnp.dot(h.reshape(BB * BLK, 32), wc_ref[...],
                 preferred_element_type=jnp.float32).reshape(BB, BLK, wo)
    dis = dis_ref[...].reshape(1, BLK, 1)
    sdeg = sdeg_ref[...].reshape(1, BLK, 1)
    y0n[...] = hw[..., 0:fo]
    qd1n[...] = sdeg * hw[..., fo:2 * fo]
    qd2n[...] = sdeg * hw[..., 2 * fo:3 * fo]
    p3n[...] = dis * hw[..., 3 * fo:4 * fo]


def _mid(y0, zz, gp, bep, dis, sdeg, br, wc):
    wo = wc.shape[1]
    fo = wo // 4
    out = jax.ShapeDtypeStruct((BB, NP, fo), jnp.float32)
    bs_o = pl.BlockSpec((BB, BLK, fo), lambda i: (0, i, 0))
    return pl.pallas_call(
        functools.partial(_mid_body, wo),
        grid=(NP // BLK,),
        in_specs=[
            pl.BlockSpec((BB, BLK, 32), lambda i: (0, i, 0)),
            pl.BlockSpec((BB, BLK, 32), lambda i: (0, i, 0)),
            pl.BlockSpec((BLK, 32), lambda i: (i, 0)),
            pl.BlockSpec((BLK, 32), lambda i: (i, 0)),
            pl.BlockSpec((BLK, 1), lambda i: (i, 0)),
            pl.BlockSpec((BLK, 1), lambda i: (i, 0)),
            pl.BlockSpec((1, 32), lambda i: (0, 0)),
            pl.BlockSpec((32, wo), lambda i: (0, 0)),
        ],
        out_specs=[bs_o, bs_o, bs_o, bs_o],
        out_shape=[out, out, out, out],
    )(y0, zz, gp, bep, dis, sdeg, br, wc)


def _final_body(y0_ref, z_ref, b_ref, vmin_ref, vmax_ref, o_ref):
    t = y0_ref[...] + z_ref[...] + b_ref[...]
    sg = jax.nn.sigmoid(t * 0.1)
    a = vmin_ref[...][None]
    b = vmax_ref[...][None]
    o_ref[...] = a + (b - a) * sg


def _final(y0, zz, b3r, vminp, vmaxp):
    return pl.pallas_call(
        _final_body,
        grid=(NP // BLK,),
        in_specs=[
            pl.BlockSpec((BB, BLK, 16), lambda i: (0, i, 0)),
            pl.BlockSpec((BB, BLK, 16), lambda i: (0, i, 0)),
            pl.BlockSpec((1, 16), lambda i: (0, 0)),
            pl.BlockSpec((BLK, 16), lambda i: (i, 0)),
            pl.BlockSpec((BLK, 16), lambda i: (i, 0)),
        ],
        out_specs=pl.BlockSpec((BB, BLK, 16), lambda i: (0, i, 0)),
        out_shape=jax.ShapeDtypeStruct((BB, NP, 16), jnp.float32),
    )(y0, zz, b3r, vminp, vmaxp)


@jax.jit
def kernel(x, edge_index, W1, b1, W2, b2, W3, b3, g1, be1, g2, be2,
           val_min, val_max):
    row = edge_index[0]
    col = edge_index[1]
    rowp = jnp.concatenate([row, jnp.full((EP - EE,), NN, jnp.int32)])
    colp = jnp.concatenate([col, jnp.full((EP - EE,), NN + 1, jnp.int32)])
    rowB = jnp.concatenate([rowp, rowp + NP])
    col1d = colp

    xf = jnp.pad(x, ((0, 0), (0, NP - NN), (0, 0))).reshape(BB * NP, 32)
    wc1 = jnp.concatenate([W1[0], W1[1], W1[2], W1[3]], axis=1)
    wc2 = jnp.concatenate([W2[0], W2[1], W2[2], W2[3]], axis=1)
    w3p = jnp.pad(W3, ((0, 0), (0, 0), (0, 13)))
    wc3 = jnp.concatenate([w3p[0], w3p[1], w3p[2], w3p[3]], axis=1)
    g1p = jnp.pad(g1.reshape(NN, 32), ((0, NP - NN), (0, 0)))
    be1p = jnp.pad(be1.reshape(NN, 32), ((0, NP - NN), (0, 0)))
    g2p = jnp.pad(g2.reshape(NN, 32), ((0, NP - NN), (0, 0)))
    be2p = jnp.pad(be2.reshape(NN, 32), ((0, NP - NN), (0, 0)))
    vminp = jnp.pad(val_min, ((0, NP - NN), (0, 13)))
    vmaxp = jnp.pad(val_max, ((0, NP - NN), (0, 13)))
    b1r = b1.reshape(1, 32)
    b2r = b2.reshape(1, 32)
    b3r = jnp.pad(b3, (0, 13)).reshape(1, 16)
    z32 = jnp.zeros((NP, 32), jnp.float32)
    z16 = jnp.zeros((NP, 16), jnp.float32)

    degp = _deg_kernel(col1d)
    y0a, qd1a, qd2a, p3a, d2x, d1x, d2x16, d1x16, dis, sdeg = _prep1(
        xf, degp.reshape(2, NP, 16), wc1)

    col2d = colp.reshape(EP // 128, 128)
    z1, _ = _hops32(p3a, qd2a, qd1a, z32, d2x, d1x, rowB, col2d)

    y0b, qd1b, qd2b, p3b = _mid(y0a.reshape(BB, NP, 32),
                                z1.reshape(BB, NP, 32),
                                g1p, be1p, dis, sdeg, b1r, wc2)
    z2, _ = _hops32(p3b.reshape(BB * NP, 32), qd2b.reshape(BB * NP, 32),
                    qd1b.reshape(BB * NP, 32), z32, d2x, d1x, rowB, col2d)

    y0c, qd1c, qd2c, p3c = _mid(y0b, z2.reshape(BB, NP, 32),
                                g2p, be2p, dis, sdeg, b2r, wc3)
    z3, _ = _hops16(p3c.reshape(BB * NP, 16), qd2c.reshape(BB * NP, 16),
                    qd1c.reshape(BB * NP, 16), z16, d2x16, d1x16, rowB, col2d)

    res = _final(y0c, z3.reshape(BB, NP, 16), b3r, vminp, vmaxp)
    return res[:, :NN, :3]


# speedup vs baseline: 1.4961x; 1.0764x over previous
import functools

import jax
import jax.numpy as jnp
from jax import lax
from jax.experimental import pallas as pl
from jax.experimental.pallas import tpu as pltpu
from jax.experimental.pallas import tpu_sc as plsc

NN = 50000
EE = 800000
BB = 2
NP = 51200
EP = 802816
NS = NP // 16
EPT = EP // 16
STE = 256
NST = EPT // STE
PCH = 64
NCH = NS // PCH
BLK = 1024

_mesh = plsc.VectorSubcoreMesh(core_axis_name="c", subcore_axis_name="s")
_params = pltpu.CompilerParams(use_tc_tiling_on_sc=False)


def _fill(ref, rows, width, val):
    v = jnp.full((16,), val, jnp.float32)

    def body(r, _):
        for h in range(width // 16):
            ref[r, pl.ds(h * 16, 16)] = v
        return 0

    lax.fori_loop(0, rows, body, 0)


def _wt(src, dst, sem):
    pltpu.make_async_copy(src, dst, sem).wait()


@functools.partial(
    pl.kernel,
    out_type=jax.ShapeDtypeStruct((2 * NP, 16), jnp.float32),
    mesh=_mesh,
    scratch_types=[
        pltpu.VMEM_SHARED((NP, 16), jnp.float32),
        pltpu.VMEM((512,), jnp.int32),
        pltpu.VMEM((512, 16), jnp.float32),
        pltpu.VMEM((PCH, 16), jnp.float32),
        pltpu.SemaphoreType.DMA,
    ],
    compiler_params=_params,
)
def _deg_kernel(col1d, degp, dacc, cbuf, ones, zb, sem):
    c = lax.axis_index("c")
    s = lax.axis_index("s")
    _fill(ones, 512, 16, 1.0)
    _fill(zb, PCH, 16, 0.0)

    def zero(i, _):
        pltpu.sync_copy(zb, dacc.at[pl.ds(s * NS + i * PCH, PCH)])
        return 0

    lax.fori_loop(0, NS // PCH, zero, 0)
    plsc.subcore_barrier()

    base = c * (EP // 2) + s * (EP // 32)

    def stage(t, _):
        pltpu.sync_copy(col1d.at[pl.ds(base + t * 512, 512)], cbuf)
        pltpu.async_copy(ones, dacc.at[cbuf], sem, add=True).wait()
        return 0

    lax.fori_loop(0, 49, stage, 0)
    plsc.subcore_barrier()

    def out(i, _):
        g = s * NS + i * PCH
        pltpu.sync_copy(dacc.at[pl.ds(g, PCH)], zb)
        pltpu.sync_copy(zb, degp.at[pl.ds(c * NP + g, PCH)])
        return 0

    lax.fori_loop(0, NS // PCH, out, 0)


CPT = EP // 128 // 16
PPC = 128


def _make_hops(width):
    halves = width // 16

    @functools.partial(
        pl.kernel,
        out_type=(
            jax.ShapeDtypeStruct((BB * NP, width), jnp.float32),
            jax.ShapeDtypeStruct((BB * NP, width), jnp.float32),
        ),
        mesh=_mesh,
        scratch_types=[
            pltpu.VMEM_SHARED((NP, width), jnp.float32),
            pltpu.VMEM((512,), jnp.int32),
            pltpu.VMEM((4, 128), jnp.int32),
            pltpu.VMEM((128, width), jnp.float32),
            pltpu.VMEM((128, width), jnp.float32),
            pltpu.VMEM((128, width), jnp.float32),
            pltpu.VMEM((128, width), jnp.float32),
            pltpu.SemaphoreType.DMA,
            pltpu.SemaphoreType.DMA,
            pltpu.SemaphoreType.DMA,
            pltpu.SemaphoreType.DMA,
            pltpu.SemaphoreType.DMA,
            pltpu.SemaphoreType.DMA,
            pltpu.SemaphoreType.DMA,
            pltpu.SemaphoreType.DMA,
            pltpu.SemaphoreType.DMA,
        ],
        compiler_params=_params,
    )
    def hops(p_init, qd2, qd1, zz, d2x, d1x, rowB, col2d, z, ptmp,
             acc, rbig, cbig, gb0, gb1, gb2, gb3,
             gs0, gs1, gs2, gs3, ss0, ss1, ss2, ss3, isem):
        c = lax.axis_index("c")
        s = lax.axis_index("s")
        nbase = s * NS
        ebase = c * EP + s * EPT
        gbs = [gb0, gb1, gb2, gb3]
        gss = [gs0, gs1, gs2, gs3]
        sss = [ss0, ss1, ss2, ss3]

        def edge_phase(tab):
            def body(i, _):
                pltpu.sync_copy(rowB.at[pl.ds(ebase + i * 512, 512)], rbig)
                pltpu.sync_copy(col2d.at[pl.ds(s * CPT + i * 4, 4)], cbig)
                hg = [
                    pltpu.async_copy(
                        tab.at[rbig.at[pl.ds(j * 128, 128)]], gbs[j], gss[j])
                    for j in range(4)
                ]
                hs = []
                for j in range(4):
                    hg[j].wait()
                    hs.append(pltpu.async_copy(
                        gbs[j], acc.at[cbig.at[j]], sss[j], add=True))
                for h in hs:
                    h.wait()
                return 0

            lax.fori_loop(0, EPT // 512, body, 0)

        def comp(ab, db):
            def rowfn(r, _):
                for h in range(halves):
                    sl = pl.ds(h * 16, 16)
                    ab[r, sl] = ab[r, sl] * db[r, sl]
                return 0

            lax.fori_loop(0, PPC, rowfn, 0)

        def post(dref, dst, initsrc, init_off):
            ioff = (c * NP + nbase) if init_off else nbase

            def chunk(i, _):
                g = nbase + i * PPC
                pltpu.async_copy(acc.at[pl.ds(g, PPC)], gb0, gs0).wait()
                pltpu.async_copy(dref.at[pl.ds(g, PPC)], gb1, gs1).wait()
                comp(gb0, gb1)
                pltpu.async_copy(
                    gb0, dst.at[pl.ds(c * NP + g, PPC)], ss0).wait()
                if initsrc is not None:
                    pltpu.async_copy(
                        initsrc.at[pl.ds(ioff + i * PPC, PPC)], gb2,
                        isem).wait()
                    pltpu.sync_copy(gb2, acc.at[pl.ds(g, PPC)])
                return 0

            lax.fori_loop(0, NS // PPC, chunk, 0)

        def init0(i, _):
            g = nbase + i * PPC
            pltpu.async_copy(
                qd2.at[pl.ds(c * NP + g, PPC)], gb2, isem).wait()
            pltpu.sync_copy(gb2, acc.at[pl.ds(g, PPC)])
            return 0

        lax.fori_loop(0, NS // PPC, init0, 0)
        plsc.subcore_barrier()
        edge_phase(p_init)
        plsc.subcore_barrier()
        post(d2x, ptmp, qd1, True)
        plsc.subcore_barrier()
        edge_phase(ptmp)
        plsc.subcore_barrier()
        post(d2x, ptmp, zz, False)
        plsc.subcore_barrier()
        edge_phase(ptmp)
        plsc.subcore_barrier()
        post(d1x, z, None, False)

    return hops


_hops32 = _make_hops(32)


@functools.partial(
    pl.kernel,
    out_type=jax.ShapeDtypeStruct((BB * NP, 16), jnp.float32),
    mesh=_mesh,
    scratch_types=[
        pltpu.VMEM_SHARED((NP, 16), jnp.float32),
        pltpu.VMEM((1024,), jnp.int32),
        pltpu.VMEM((1024,), jnp.int32),
        pltpu.VMEM((1024,), jnp.int32),
        pltpu.VMEM((1024,), jnp.int32),
        pltpu.VMEM((1024, 16), jnp.float32),
        pltpu.VMEM((1024, 16), jnp.float32),
        pltpu.VMEM((PPC, 16), jnp.float32),
        pltpu.VMEM((PPC, 16), jnp.float32),
        pltpu.VMEM((PPC, 16), jnp.float32),
        pltpu.VMEM((PPC, 16), jnp.float32),
        pltpu.VMEM((1, 16), jnp.float32),
        pltpu.SemaphoreType.DMA,
        pltpu.SemaphoreType.DMA,
        pltpu.SemaphoreType.DMA,
        pltpu.SemaphoreType.DMA,
        pltpu.SemaphoreType.DMA,
    ],
    compiler_params=_params,
)
def _hops16(p_init, qd2, qd1, zz, d2x, d1x, y0, b3r, vminp, vmaxp,
            rowB, col1d, out,
            acc, r0, c0, r1, c1, gb0, gb1, aA, dA, xA, xB, bv,
            gs0, gs1, ss0, ss1, isem):
    c = lax.axis_index("c")
    s = lax.axis_index("s")
    nbase = s * NS
    ebase = c * EP + s * EPT
    cbase = s * EPT

    pltpu.sync_copy(b3r, bv)

    def edge_phase(tab):
        def body(i, _):
            t0 = i * 2048
            pltpu.sync_copy(rowB.at[pl.ds(ebase + t0, 1024)], r0)
            hg0 = pltpu.async_copy(tab.at[r0], gb0, gs0)
            pltpu.sync_copy(rowB.at[pl.ds(ebase + t0 + 1024, 1024)], r1)
            hg1 = pltpu.async_copy(tab.at[r1], gb1, gs1)
            pltpu.sync_copy(col1d.at[pl.ds(cbase + t0, 1024)], c0)
            pltpu.sync_copy(col1d.at[pl.ds(cbase + t0 + 1024, 1024)], c1)
            hg0.wait()
            hs0 = pltpu.async_copy(gb0, acc.at[c0], ss0, add=True)
            hg1.wait()
            hs1 = pltpu.async_copy(gb1, acc.at[c1], ss1, add=True)
            hs0.wait()
            hs1.wait()
            return 0

        lax.fori_loop(0, EPT // 2048, body, 0)

    def comp(ab, db):
        def rowfn(r, _):
            sl = pl.ds(0, 16)
            ab[r, sl] = ab[r, sl] * db[r, sl]
            return 0

        lax.fori_loop(0, PPC, rowfn, 0)

    def post(dref, dst, initsrc, init_off):
        ioff = (c * NP + nbase) if init_off else nbase

        def chunk(i, _):
            g = nbase + i * PPC
            pltpu.async_copy(acc.at[pl.ds(g, PPC)], aA, gs0).wait()
            pltpu.async_copy(dref.at[pl.ds(g, PPC)], dA, gs1).wait()
            comp(aA, dA)
            pltpu.async_copy(aA, dst.at[pl.ds(c * NP + g, PPC)], ss0).wait()
            if initsrc is not None:
                pltpu.async_copy(
                    initsrc.at[pl.ds(ioff + i * PPC, PPC)], xA, isem).wait()
                pltpu.sync_copy(xA, acc.at[pl.ds(g, PPC)])
            return 0

        lax.fori_loop(0, NS // PPC, chunk, 0)

    def post_final():
        b3 = bv[0, pl.ds(0, 16)]

        def chunk(i, _):
            g = nbase + i * PPC
            pltpu.async_copy(acc.at[pl.ds(g, PPC)], aA, gs0).wait()
            pltpu.async_copy(d1x.at[pl.ds(g, PPC)], dA, gs1).wait()
            pltpu.async_copy(y0.at[pl.ds(c * NP + g, PPC)], xA, isem).wait()
            pltpu.async_copy(vminp.at[pl.ds(g, PPC)], xB, ss1).wait()

            def rowfn(r, _):
                sl = pl.ds(0, 16)
                t = (xA[r, sl] + aA[r, sl] * dA[r, sl] + b3) * 0.1
                sg = 1.0 / (1.0 + jnp.exp(-t))
                aA[r, sl] = sg
                return 0

            lax.fori_loop(0, PPC, rowfn, 0)
            pltpu.async_copy(vmaxp.at[pl.ds(g, PPC)], dA, ss1).wait()

            def rowfn2(r, _):
                sl = pl.ds(0, 16)
                aA[r, sl] = xB[r, sl] + (dA[r, sl] - xB[r, sl]) * aA[r, sl]
                return 0

            lax.fori_loop(0, PPC, rowfn2, 0)
            pltpu.async_copy(aA, out.at[pl.ds(c * NP + g, PPC)], ss0).wait()
            return 0

        lax.fori_loop(0, NS // PPC, chunk, 0)

    def init0(i, _):
        g = nbase + i * PPC
        pltpu.async_copy(qd2.at[pl.ds(c * NP + g, PPC)], xA, isem).wait()
        pltpu.sync_copy(xA, acc.at[pl.ds(g, PPC)])
        return 0

    lax.fori_loop(0, NS // PPC, init0, 0)
    plsc.subcore_barrier()
    edge_phase(p_init)
    plsc.subcore_barrier()
    post(d2x, out, qd1, True)
    plsc.subcore_barrier()
    edge_phase(out)
    plsc.subcore_barrier()
    post(d2x, out, zz, False)
    plsc.subcore_barrier()
    edge_phase(out)
    plsc.subcore_barrier()
    post_final()


def _prep1_body(x_ref, degp_ref, wc_ref,
                y0, qd1, qd2, p3, d2x, d1x, d2x16, d1x16, dis_o, sdeg_o):
    deg = degp_ref[0, :, 0] + degp_ref[1, :, 0]
    dis = jnp.where(deg > 0, lax.rsqrt(jnp.maximum(deg, 1e-12)), 0.0)
    dis = dis.reshape(BLK, 1)
    sdeg = jnp.sqrt(deg).reshape(BLK, 1)
    xw = jnp.dot(x_ref[...], wc_ref[...], preferred_element_type=jnp.float32)
    y0[...] = xw[:, 0:32]
    qd1[...] = sdeg * xw[:, 32:64]
    qd2[...] = sdeg * xw[:, 64:96]
    p3[...] = dis * xw[:, 96:128]
    d2 = dis * dis
    d2x[...] = jnp.broadcast_to(d2, (BLK, 32))
    d1x[...] = jnp.broadcast_to(dis, (BLK, 32))
    d2x16[...] = jnp.broadcast_to(d2, (BLK, 16))
    d1x16[...] = jnp.broadcast_to(dis, (BLK, 16))
    dis_o[...] = dis
    sdeg_o[...] = sdeg


def _prep1(xf, degp2, wc1):
    nb = NP // BLK
    fo32 = jax.ShapeDtypeStruct((BB * NP, 32), jnp.float32)
    n32 = jax.ShapeDtypeStruct((NP, 32), jnp.float32)
    n16 = jax.ShapeDtypeStruct((NP, 16), jnp.float32)
    n1 = jax.ShapeDtypeStruct((NP, 1), jnp.float32)
    bs_bn = pl.BlockSpec((BLK, 32), lambda i: (i, 0))
    bs_n32 = pl.BlockSpec((BLK, 32), lambda i: (i % nb, 0))
    bs_n16 = pl.BlockSpec((BLK, 16), lambda i: (i % nb, 0))
    bs_n1 = pl.BlockSpec((BLK, 1), lambda i: (i % nb, 0))
    return pl.pallas_call(
        _prep1_body,
        grid=(BB * NP // BLK,),
        in_specs=[
            bs_bn,
            pl.BlockSpec((2, BLK, 16), lambda i: (0, i % nb, 0)),
            pl.BlockSpec((32, 128), lambda i: (0, 0)),
        ],
        out_specs=[bs_bn, bs_bn, bs_bn, bs_bn,
                   bs_n32, bs_n32, bs_n16, bs_n16, bs_n1, bs_n1],
        out_shape=[fo32, fo32, fo32, fo32, n32, n32, n16, n16, n1, n1],
    )(xf, degp2, wc1)


def _mid_body(wo, y0_ref, z_ref, g_ref, be_ref, dis_ref, sdeg_ref, b_ref,
              wc_ref, y0n, qd1n, qd2n, p3n):
    fo = wo // 4
    t = y0_ref[...] + z_ref[...] + b_ref[...]
    m = jnp.mean(t, axis=0, keepdims=True)
    v = jnp.mean((t - m) ** 2, axis=0, keepdims=True)
    h = g_ref[...] * (t - m) / jnp.sqrt(v + 1e-5) + be_ref[...]
    h = jnp.where(h >= 0, h, 0.01 * h)
    hw = jnp.dot(h.reshape(BB * BLK, 32), wc_ref[...],
                 preferred_element_type=jnp.float32).reshape(BB, BLK, wo)
    dis = dis_ref[...].reshape(1, BLK, 1)
    sdeg = sdeg_ref[...].reshape(1, BLK, 1)
    y0n[...] = hw[..., 0:fo]
    qd1n[...] = sdeg * hw[..., fo:2 * fo]
    qd2n[...] = sdeg * hw[..., 2 * fo:3 * fo]
    p3n[...] = dis * hw[..., 3 * fo:4 * fo]


def _mid(y0, zz, gp, bep, dis, sdeg, br, wc):
    wo = wc.shape[1]
    fo = wo // 4
    out = jax.ShapeDtypeStruct((BB, NP, fo), jnp.float32)
    bs_o = pl.BlockSpec((BB, BLK, fo), lambda i: (0, i, 0))
    return pl.pallas_call(
        functools.partial(_mid_body, wo),
        grid=(NP // BLK,),
        in_specs=[
            pl.BlockSpec((BB, BLK, 32), lambda i: (0, i, 0)),
            pl.BlockSpec((BB, BLK, 32), lambda i: (0, i, 0)),
            pl.BlockSpec((BLK, 32), lambda i: (i, 0)),
            pl.BlockSpec((BLK, 32), lambda i: (i, 0)),
            pl.BlockSpec((BLK, 1), lambda i: (i, 0)),
            pl.BlockSpec((BLK, 1), lambda i: (i, 0)),
            pl.BlockSpec((1, 32), lambda i: (0, 0)),
            pl.BlockSpec((32, wo), lambda i: (0, 0)),
        ],
        out_specs=[bs_o, bs_o, bs_o, bs_o],
        out_shape=[out, out, out, out],
    )(y0, zz, gp, bep, dis, sdeg, br, wc)


def _final_body(y0_ref, z_ref, b_ref, vmin_ref, vmax_ref, o_ref):
    t = y0_ref[...] + z_ref[...] + b_ref[...]
    sg = jax.nn.sigmoid(t * 0.1)
    a = vmin_ref[...][None]
    b = vmax_ref[...][None]
    o_ref[...] = a + (b - a) * sg


def _final(y0, zz, b3r, vminp, vmaxp):
    return pl.pallas_call(
        _final_body,
        grid=(NP // BLK,),
        in_specs=[
            pl.BlockSpec((BB, BLK, 16), lambda i: (0, i, 0)),
            pl.BlockSpec((BB, BLK, 16), lambda i: (0, i, 0)),
            pl.BlockSpec((1, 16), lambda i: (0, 0)),
            pl.BlockSpec((BLK, 16), lambda i: (i, 0)),
            pl.BlockSpec((BLK, 16), lambda i: (i, 0)),
        ],
        out_specs=pl.BlockSpec((BB, BLK, 16), lambda i: (0, i, 0)),
        out_shape=jax.ShapeDtypeStruct((BB, NP, 16), jnp.float32),
    )(y0, zz, b3r, vminp, vmaxp)


@jax.jit
def kernel(x, edge_index, W1, b1, W2, b2, W3, b3, g1, be1, g2, be2,
           val_min, val_max):
    row = edge_index[0]
    col = edge_index[1]
    rowp = jnp.concatenate([row, jnp.full((EP - EE,), NN, jnp.int32)])
    colp = jnp.concatenate([col, jnp.full((EP - EE,), NN + 1, jnp.int32)])
    rowB = jnp.concatenate([rowp, rowp + NP])
    col1d = colp

    xf = jnp.pad(x, ((0, 0), (0, NP - NN), (0, 0))).reshape(BB * NP, 32)
    wc1 = jnp.concatenate([W1[0], W1[1], W1[2], W1[3]], axis=1)
    wc2 = jnp.concatenate([W2[0], W2[1], W2[2], W2[3]], axis=1)
    w3p = jnp.pad(W3, ((0, 0), (0, 0), (0, 13)))
    wc3 = jnp.concatenate([w3p[0], w3p[1], w3p[2], w3p[3]], axis=1)
    g1p = jnp.pad(g1.reshape(NN, 32), ((0, NP - NN), (0, 0)))
    be1p = jnp.pad(be1.reshape(NN, 32), ((0, NP - NN), (0, 0)))
    g2p = jnp.pad(g2.reshape(NN, 32), ((0, NP - NN), (0, 0)))
    be2p = jnp.pad(be2.reshape(NN, 32), ((0, NP - NN), (0, 0)))
    vminp = jnp.pad(val_min, ((0, NP - NN), (0, 13)))
    vmaxp = jnp.pad(val_max, ((0, NP - NN), (0, 13)))
    b1r = b1.reshape(1, 32)
    b2r = b2.reshape(1, 32)
    b3r = jnp.pad(b3, (0, 13)).reshape(1, 16)
    z32 = jnp.zeros((NP, 32), jnp.float32)
    z16 = jnp.zeros((NP, 16), jnp.float32)

    degp = _deg_kernel(col1d)
    y0a, qd1a, qd2a, p3a, d2x, d1x, d2x16, d1x16, dis, sdeg = _prep1(
        xf, degp.reshape(2, NP, 16), wc1)

    col2d = colp.reshape(EP // 128, 128)
    z1, _ = _hops32(p3a, qd2a, qd1a, z32, d2x, d1x, rowB, col2d)

    y0b, qd1b, qd2b, p3b = _mid(y0a.reshape(BB, NP, 32),
                                z1.reshape(BB, NP, 32),
                                g1p, be1p, dis, sdeg, b1r, wc2)
    z2, _ = _hops32(p3b.reshape(BB * NP, 32), qd2b.reshape(BB * NP, 32),
                    qd1b.reshape(BB * NP, 32), z32, d2x, d1x, rowB, col2d)

    y0c, qd1c, qd2c, p3c = _mid(y0b, z2.reshape(BB, NP, 32),
                                g2p, be2p, dis, sdeg, b2r, wc3)
    res = _hops16(p3c.reshape(BB * NP, 16), qd2c.reshape(BB * NP, 16),
                  qd1c.reshape(BB * NP, 16), z16, d2x16, d1x16,
                  y0c.reshape(BB * NP, 16), b3r, vminp, vmaxp, rowB, col1d)

    return res.reshape(BB, NP, 16)[:, :NN, :3]
